# baseline (speedup 1.0000x reference)
import jax
import jax.numpy as jnp
from jax.experimental import pallas as pl
from jax.experimental.pallas import tpu as pltpu

_SEED = 512


def _body(a_ref, o_hbm, buf, sems):
    buf[...] = jnp.broadcast_to(a_ref[...], buf.shape)
    B = o_hbm.shape[0]
    n = B // _SEED
    seed = pltpu.make_async_copy(buf, o_hbm.at[pl.ds(0, _SEED), :], sems.at[0])
    seed.start()
    seed.wait()
    copies = [
        pltpu.make_async_copy(
            o_hbm.at[pl.ds(0, _SEED), :],
            o_hbm.at[pl.ds(k * _SEED, _SEED), :],
            sems.at[k],
        )
        for k in range(1, n)
    ]
    for c in copies:
        c.start()
    for c in copies:
        c.wait()


def kernel(x, action):
    B = x.shape[0]
    A = action.shape[0]
    a2 = action.reshape(1, A)
    return pl.pallas_call(
        _body,
        in_specs=[pl.BlockSpec((1, A), lambda: (0, 0))],
        out_specs=pl.BlockSpec(memory_space=pl.ANY),
        out_shape=jax.ShapeDtypeStruct((B, A), jnp.float32),
        scratch_shapes=[
            pltpu.VMEM((_SEED, A), jnp.float32),
            pltpu.SemaphoreType.DMA((B // _SEED,)),
        ],
    )(a2)


# speedup vs baseline: 80.5828x; 80.5828x over previous
import jax
import jax.numpy as jnp
from jax.experimental import pallas as pl


def _body(a_ref, o_ref):
    o_ref[...] = jnp.broadcast_to(a_ref[...], o_ref.shape)


def kernel(x, action):
    B = x.shape[0]
    A = action.shape[0]
    a2 = action.reshape(A, 1)
    wide = pl.pallas_call(
        _body,
        in_specs=[pl.BlockSpec((A, 1), lambda: (0, 0))],
        out_specs=pl.BlockSpec((A, B), lambda: (0, 0)),
        out_shape=jax.ShapeDtypeStruct((A, B), jnp.float32),
    )(a2)
    return wide.T


# speedup vs baseline: 147.1999x; 1.8267x over previous
import jax
import jax.numpy as jnp
from jax.experimental import pallas as pl


def _body(a_ref, o_ref):
    col = a_ref[...].reshape(a_ref.shape[1], 1)
    o_ref[...] = jnp.broadcast_to(col, o_ref.shape)


def kernel(x, action):
    B = x.shape[0]
    A = action.shape[0]
    a2 = action.reshape(1, A)
    wide = pl.pallas_call(
        _body,
        in_specs=[pl.BlockSpec((1, A), lambda: (0, 0))],
        out_specs=pl.BlockSpec((A, B), lambda: (0, 0)),
        out_shape=jax.ShapeDtypeStruct((A, B), jnp.float32),
    )(a2)
    return wide.T
